# Initial kernel scaffold; baseline (speedup 1.0000x reference)
import jax
import jax.numpy as jnp
from jax.experimental import pallas as pl


def kernel(node_emb, node_type, node_degree, pos, linker_mask, fragment_mask, edge_index, edge_type, batch, time_step, deg_emb, et_emb, el_W1, el_b1, el_W2, el_b2, gin_W1, gin_b1, gin_W2, gin_b2, gm_W1, gm_b1, gm_W2, gm_b2, gm_W3, gm_b3):
    raise NotImplementedError("write your pallas kernel here")


# speedup vs baseline: 1.1155x; 1.1155x over previous
import functools

import jax
import jax.numpy as jnp
from jax.experimental import pallas as pl
from jax.experimental.pallas import tpu as pltpu

N = 10000
E = 160000
H = 128
NB = 2
NC = 4

EBLK = 6400


def _bf16_dot(a, b):
    return jnp.dot(a.astype(jnp.bfloat16), b.astype(jnp.bfloat16),
                   preferred_element_type=jnp.float32)


def _edge_encoder_body(el_ref, et_ref, elW1_ref, elb1_ref, elW2_ref, elb2_ref,
                       etemb_ref, attr_ref):
    el = el_ref[...]
    eh = jnp.maximum(el * elW1_ref[...] + elb1_ref[...], 0.0)
    eh = _bf16_dot(eh, elW2_ref[...]) + elb2_ref[...]
    et = et_ref[0, 0]
    onehot = (et[:, None] == jax.lax.broadcasted_iota(jnp.int32, (1, 8), 1)
              ).astype(jnp.float32)
    emb = jax.lax.dot(onehot, etemb_ref[...],
                      precision=jax.lax.Precision.HIGHEST)
    attr_ref[...] = eh * emb


def _edge_encoder(edge_length, edge_type, el_W1, el_b1, el_W2, el_b2, et_emb):
    nblk = E // EBLK
    et3 = edge_type.reshape(nblk, 1, EBLK).astype(jnp.int32)
    return pl.pallas_call(
        _edge_encoder_body,
        grid=(nblk,),
        in_specs=[
            pl.BlockSpec((EBLK, 1), lambda i: (i, 0)),
            pl.BlockSpec((1, 1, EBLK), lambda i: (i, 0, 0)),
            pl.BlockSpec((1, H), lambda i: (0, 0)),
            pl.BlockSpec((H,), lambda i: (0,)),
            pl.BlockSpec((H, H), lambda i: (0, 0)),
            pl.BlockSpec((H,), lambda i: (0,)),
            pl.BlockSpec((8, H), lambda i: (0, 0)),
        ],
        out_specs=pl.BlockSpec((EBLK, H), lambda i: (i, 0)),
        out_shape=jax.ShapeDtypeStruct((E, H), jnp.float32),
    )(edge_length, et3, el_W1, el_b1, el_W2, el_b2, et_emb)


def kernel(node_emb, node_type, node_degree, pos, linker_mask, fragment_mask,
           edge_index, edge_type, batch, time_step,
           deg_emb, et_emb, el_W1, el_b1, el_W2, el_b2,
           gin_W1, gin_b1, gin_W2, gin_b2,
           gm_W1, gm_b1, gm_W2, gm_b2, gm_W3, gm_b3):
    row, col = edge_index[0], edge_index[1]
    d0 = pos[row] - pos[col]
    edge_length = jnp.sqrt(jnp.sum(d0 * d0, axis=-1, keepdims=True) + 1e-12)
    edge_attr = _edge_encoder(edge_length, edge_type,
                              el_W1, el_b1, el_W2, el_b2, et_emb)
    z = jnp.take(deg_emb, node_degree, axis=0)
    W1a = gm_W1[:H]
    W1b = gm_W1[H:2 * H]
    W1c = gm_W1[2 * H:]
    C = edge_attr @ W1c + gm_b1
    pos0 = pos
    pf = pos
    for b in range(NB):
        h = z
        for c in range(NC):
            msg = jax.nn.relu(h[row] + edge_attr)
            agg = jax.ops.segment_sum(msg, col, num_segments=N)
            h_in = h + agg
            h_out = jax.nn.relu(h_in @ gin_W1[b, c] + gin_b1[b, c]) @ gin_W2[b, c] + gin_b2[b, c]
            h = h + jax.nn.relu(h_out)
        A = h @ W1a
        B = h @ W1b
        e1 = jax.nn.relu(A[row] + B[col] + C)
        e2 = jax.nn.relu(e1 @ gm_W2 + gm_b2)
        edge_inv = e2 @ gm_W3 + gm_b3
        dd_dr = (pf[row] - pf[col]) / edge_length
        upd = dd_dr * edge_inv
        node_eq = (jax.ops.segment_sum(upd, row, num_segments=N)
                   - jax.ops.segment_sum(upd, col, num_segments=N))
        pf = pf + node_eq
        pf = pf * linker_mask + pos0 * fragment_mask
    return pf - pos0


# speedup vs baseline: 1.4288x; 1.2808x over previous
import functools

import jax
import jax.numpy as jnp
from jax import lax
from jax.experimental import pallas as pl
from jax.experimental.pallas import tpu as pltpu
import jax.experimental.pallas.tpu_sc as plsc

N = 10000
E = 160000
H = 128
NB = 2
NC = 4

K = 128
CH = 40
E_PAD = 2 * 16 * CH * K
PAD = E_PAD - E
AGG_ROWS = 10240
NEG = -1e9

EBLK = 8192
BN = 2000


def _bf16_dot(a, b):
    return jnp.dot(a.astype(jnp.bfloat16), b.astype(jnp.bfloat16),
                   preferred_element_type=jnp.float32)


def _edge_encoder_body(el_ref, et_ref, elW1_ref, elb1_ref, elW2_ref, elb2_ref,
                       etemb_ref, attr_ref):
    el = el_ref[...]
    eh = jnp.maximum(el * elW1_ref[...] + elb1_ref[...], 0.0)
    eh = _bf16_dot(eh, elW2_ref[...]) + elb2_ref[...]
    et = et_ref[0, 0]
    onehot = (et[:, None] == lax.broadcasted_iota(jnp.int32, (1, 8), 1)
              ).astype(jnp.float32)
    emb = lax.dot(onehot, etemb_ref[...],
                  precision=lax.Precision.HIGHEST)
    attr = eh * emb
    gid = pl.program_id(0) * EBLK + lax.broadcasted_iota(jnp.int32, (EBLK, 1), 0)
    attr_ref[...] = jnp.where(gid < E, attr, NEG)


def _edge_encoder(el_pad, et_pad, el_W1, el_b1, el_W2, el_b2, et_emb):
    nblk = E_PAD // EBLK
    et3 = et_pad.reshape(nblk, 1, EBLK)
    return pl.pallas_call(
        _edge_encoder_body,
        grid=(nblk,),
        in_specs=[
            pl.BlockSpec((EBLK, 1), lambda i: (i, 0)),
            pl.BlockSpec((1, 1, EBLK), lambda i: (i, 0, 0)),
            pl.BlockSpec((1, H), lambda i: (0, 0)),
            pl.BlockSpec((H,), lambda i: (0,)),
            pl.BlockSpec((H, H), lambda i: (0, 0)),
            pl.BlockSpec((H,), lambda i: (0,)),
            pl.BlockSpec((8, H), lambda i: (0, 0)),
        ],
        out_specs=pl.BlockSpec((EBLK, H), lambda i: (i, 0)),
        out_shape=jax.ShapeDtypeStruct((E_PAD, H), jnp.float32),
    )(el_pad, et3, el_W1, el_b1, el_W2, el_b2, et_emb)


def _sc_conv_agg(h, attr_pad, row3, col3):
    @functools.partial(
        pl.kernel,
        out_type=jax.ShapeDtypeStruct((2, AGG_ROWS, H), jnp.float32),
        mesh=plsc.VectorSubcoreMesh(core_axis_name="c", subcore_axis_name="s"),
        scratch_types=[
            pltpu.VMEM((K,), jnp.int32),
            pltpu.VMEM((K,), jnp.int32),
            pltpu.VMEM((K, H), jnp.float32),
            pltpu.VMEM((K, H), jnp.float32),
            pltpu.VMEM_SHARED((AGG_ROWS, H), jnp.float32),
            pltpu.SemaphoreType.DMA,
        ],
    )
    def k(h_hbm, attr_hbm, row_hbm, col_hbm, out_hbm,
          idx_r, idx_c, rows_v, attr_v, agg_s, sem):
        cid = lax.axis_index("c")
        sid = lax.axis_index("s")
        zero = jnp.zeros((16,), jnp.float32)

        def zbody(i, carry):
            for l in range(8):
                rows_v[i, pl.ds(l * 16, 16)] = zero
            return carry
        lax.fori_loop(0, K, zbody, 0)
        for kk in range(5):
            pltpu.sync_copy(rows_v, agg_s.at[pl.ds(sid * 640 + kk * K, K)])
        plsc.subcore_barrier()

        def body(j, carry):
            pltpu.sync_copy(row_hbm.at[cid, sid, j], idx_r)
            pltpu.sync_copy(col_hbm.at[cid, sid, j], idx_c)
            pltpu.async_copy(h_hbm.at[idx_r], rows_v, sem).wait()
            g = (cid * 16 + sid) * CH + j
            pltpu.sync_copy(attr_hbm.at[pl.ds(g * K, K)], attr_v)

            def mbody(i, c2):
                for l in range(8):
                    s = pl.ds(l * 16, 16)
                    rows_v[i, s] = jnp.maximum(rows_v[i, s] + attr_v[i, s], 0.0)
                return c2
            lax.fori_loop(0, K, mbody, 0)
            pltpu.sync_copy(rows_v, agg_s.at[idx_c], add=True)
            return carry
        lax.fori_loop(0, CH, body, 0)
        plsc.subcore_barrier()
        pltpu.sync_copy(agg_s.at[pl.ds(sid * 640, 640)],
                        out_hbm.at[cid, pl.ds(sid * 640, 640)])

    return k(h, attr_pad, row3, col3)


def _conv_mlp_body(h_ref, agg_ref, W1_ref, b1_ref, W2_ref, b2_ref, out_ref):
    h = h_ref[...]
    h_in = h + (agg_ref[0] + agg_ref[1])
    t = jnp.maximum(_bf16_dot(h_in, W1_ref[...]) + b1_ref[...], 0.0)
    h_out = _bf16_dot(t, W2_ref[...]) + b2_ref[...]
    out_ref[...] = h + jnp.maximum(h_out, 0.0)


def _conv_mlp(h, agg, W1, b1, W2, b2):
    return pl.pallas_call(
        _conv_mlp_body,
        grid=(N // BN,),
        in_specs=[
            pl.BlockSpec((BN, H), lambda i: (i, 0)),
            pl.BlockSpec((2, BN, H), lambda i: (0, i, 0)),
            pl.BlockSpec((H, H), lambda i: (0, 0)),
            pl.BlockSpec((H,), lambda i: (0,)),
            pl.BlockSpec((H, H), lambda i: (0, 0)),
            pl.BlockSpec((H,), lambda i: (0,)),
        ],
        out_specs=pl.BlockSpec((BN, H), lambda i: (i, 0)),
        out_shape=jax.ShapeDtypeStruct((N, H), jnp.float32),
    )(h, agg, W1, b1, W2, b2)


def kernel(node_emb, node_type, node_degree, pos, linker_mask, fragment_mask,
           edge_index, edge_type, batch, time_step,
           deg_emb, et_emb, el_W1, el_b1, el_W2, el_b2,
           gin_W1, gin_b1, gin_W2, gin_b2,
           gm_W1, gm_b1, gm_W2, gm_b2, gm_W3, gm_b3):
    row = edge_index[0].astype(jnp.int32)
    col = edge_index[1].astype(jnp.int32)
    row3 = jnp.concatenate([row, jnp.zeros((PAD,), jnp.int32)]
                           ).reshape(2, 16, CH, K)
    col3 = jnp.concatenate([col, jnp.zeros((PAD,), jnp.int32)]
                           ).reshape(2, 16, CH, K)

    d0 = pos[row] - pos[col]
    edge_length = jnp.sqrt(jnp.sum(d0 * d0, axis=-1, keepdims=True) + 1e-12)
    el_pad = jnp.concatenate([edge_length, jnp.zeros((PAD, 1), jnp.float32)])
    et_pad = jnp.concatenate([edge_type.astype(jnp.int32),
                              jnp.zeros((PAD,), jnp.int32)])
    edge_attr = _edge_encoder(el_pad, et_pad, el_W1, el_b1, el_W2, el_b2, et_emb)

    z = jnp.take(deg_emb, node_degree, axis=0)
    W1a = gm_W1[:H]
    W1b = gm_W1[H:2 * H]
    W1c = gm_W1[2 * H:]
    C = edge_attr[:E] @ W1c + gm_b1
    pos0 = pos
    pf = pos
    for b in range(NB):
        h = z
        for c in range(NC):
            agg = _sc_conv_agg(h, edge_attr, row3, col3)
            h = _conv_mlp(h, agg, gin_W1[b, c], gin_b1[b, c],
                          gin_W2[b, c], gin_b2[b, c])
        A = h @ W1a
        B = h @ W1b
        e1 = jax.nn.relu(A[row] + B[col] + C)
        e2 = jax.nn.relu(e1 @ gm_W2 + gm_b2)
        edge_inv = e2 @ gm_W3 + gm_b3
        dd_dr = (pf[row] - pf[col]) / edge_length
        upd = dd_dr * edge_inv
        node_eq = (jax.ops.segment_sum(upd, row, num_segments=N)
                   - jax.ops.segment_sum(upd, col, num_segments=N))
        pf = pf + node_eq
        pf = pf * linker_mask + pos0 * fragment_mask
    return pf - pos0


# speedup vs baseline: 1.6154x; 1.1306x over previous
import functools

import jax
import jax.numpy as jnp
from jax import lax
from jax.experimental import pallas as pl
from jax.experimental.pallas import tpu as pltpu
import jax.experimental.pallas.tpu_sc as plsc

N = 10000
E = 160000
H = 128
NB = 2
NC = 4

K = 64
CH = 80
E_PAD = 2 * 16 * CH * K
PAD = E_PAD - E
AGG_ROWS = 10240
NEG = -1e9

EBLK = 8192
BN = 2000


def _bf16_dot(a, b):
    return jnp.dot(a.astype(jnp.bfloat16), b.astype(jnp.bfloat16),
                   preferred_element_type=jnp.float32)


def _edge_encoder_body(el_ref, et_ref, elW1_ref, elb1_ref, elW2_ref, elb2_ref,
                       etemb_ref, attr_ref):
    el = el_ref[...]
    eh = jnp.maximum(el * elW1_ref[...] + elb1_ref[...], 0.0)
    eh = _bf16_dot(eh, elW2_ref[...]) + elb2_ref[...]
    et = et_ref[0, 0]
    onehot = (et[:, None] == lax.broadcasted_iota(jnp.int32, (1, 8), 1)
              ).astype(jnp.float32)
    emb = lax.dot(onehot, etemb_ref[...],
                  precision=lax.Precision.HIGHEST)
    attr = eh * emb
    gid = pl.program_id(0) * EBLK + lax.broadcasted_iota(jnp.int32, (EBLK, 1), 0)
    attr_ref[...] = jnp.where(gid < E, attr, NEG)


def _edge_encoder(el_pad, et_pad, el_W1, el_b1, el_W2, el_b2, et_emb):
    nblk = E_PAD // EBLK
    et3 = et_pad.reshape(nblk, 1, EBLK)
    return pl.pallas_call(
        _edge_encoder_body,
        grid=(nblk,),
        in_specs=[
            pl.BlockSpec((EBLK, 1), lambda i: (i, 0)),
            pl.BlockSpec((1, 1, EBLK), lambda i: (i, 0, 0)),
            pl.BlockSpec((1, H), lambda i: (0, 0)),
            pl.BlockSpec((H,), lambda i: (0,)),
            pl.BlockSpec((H, H), lambda i: (0, 0)),
            pl.BlockSpec((H,), lambda i: (0,)),
            pl.BlockSpec((8, H), lambda i: (0, 0)),
        ],
        out_specs=pl.BlockSpec((EBLK, H), lambda i: (i, 0)),
        out_shape=jax.ShapeDtypeStruct((E_PAD, H), jnp.float32),
    )(el_pad, et3, el_W1, el_b1, el_W2, el_b2, et_emb)


def _sc_conv_agg(h, attr_pad, packed3):
    @functools.partial(
        pl.kernel,
        out_type=jax.ShapeDtypeStruct((2, AGG_ROWS, H), jnp.float32),
        mesh=plsc.VectorSubcoreMesh(core_axis_name="c", subcore_axis_name="s"),
        scratch_types=[
            pltpu.VMEM((CH, K), jnp.int32),
            pltpu.VMEM((2, K), jnp.int32),
            pltpu.VMEM((2, K), jnp.int32),
            pltpu.VMEM((2, K, H), jnp.float32),
            pltpu.VMEM((2, K, H), jnp.float32),
            pltpu.VMEM_SHARED((AGG_ROWS, H), jnp.float32),
            pltpu.SemaphoreType.DMA,
            pltpu.SemaphoreType.DMA,
            pltpu.SemaphoreType.DMA,
            pltpu.SemaphoreType.DMA,
        ],
    )
    def k(h_hbm, attr_hbm, packed_hbm, out_hbm,
          packed_v, idx_r, idx_c, rows_v, attr_v, agg_s, g0, g1, a0, a1):
        cid = lax.axis_index("c")
        sid = lax.axis_index("s")
        wbase = (cid * 16 + sid) * CH
        sem_g = (g0, g1)
        sem_a = (a0, a1)
        zero = jnp.zeros((16,), jnp.float32)

        def zbody(i, carry):
            for l in range(8):
                rows_v[0, i, pl.ds(l * 16, 16)] = zero
            return carry
        lax.fori_loop(0, K, zbody, 0)
        for kk in range(640 // K):
            pltpu.sync_copy(rows_v.at[0], agg_s.at[pl.ds(sid * 640 + kk * K, K)])
        plsc.subcore_barrier()

        pltpu.sync_copy(packed_hbm.at[cid, sid], packed_v)

        def start(j, b):
            for l in range(K // 16):
                s = pl.ds(l * 16, 16)
                p = packed_v[j, s]
                idx_r[b, s] = jnp.bitwise_and(p, 0xFFFF)
                idx_c[b, s] = lax.shift_right_logical(p, 16)
            pltpu.async_copy(h_hbm.at[idx_r.at[b]], rows_v.at[b], sem_g[b])
            pltpu.async_copy(attr_hbm.at[pl.ds((wbase + j) * K, K)],
                             attr_v.at[b], sem_a[b])

        start(0, 0)
        start(1, 1)

        def body(j2, carry):
            for b in range(2):
                j = j2 * 2 + b
                pltpu.make_async_copy(h_hbm.at[idx_r.at[b]],
                                      rows_v.at[b], sem_g[b]).wait()
                pltpu.make_async_copy(attr_hbm.at[pl.ds((wbase + j) * K, K)],
                                      attr_v.at[b], sem_a[b]).wait()

                def mbody(i, c2):
                    for l in range(8):
                        s = pl.ds(l * 16, 16)
                        rows_v[b, i, s] = jnp.maximum(
                            rows_v[b, i, s] + attr_v[b, i, s], 0.0)
                    return c2
                lax.fori_loop(0, K, mbody, 0)
                pltpu.sync_copy(rows_v.at[b], agg_s.at[idx_c.at[b]], add=True)

                @pl.when(j + 2 < CH)
                def _():
                    start(j + 2, b)
            return carry
        lax.fori_loop(0, CH // 2, body, 0)
        plsc.subcore_barrier()
        pltpu.sync_copy(agg_s.at[pl.ds(sid * 640, 640)],
                        out_hbm.at[cid, pl.ds(sid * 640, 640)])

    return k(h, attr_pad, packed3)


def _conv_mlp_body(h_ref, agg_ref, W1_ref, b1_ref, W2_ref, b2_ref, out_ref):
    h = h_ref[...]
    h_in = h + (agg_ref[0] + agg_ref[1])
    t = jnp.maximum(_bf16_dot(h_in, W1_ref[...]) + b1_ref[...], 0.0)
    h_out = _bf16_dot(t, W2_ref[...]) + b2_ref[...]
    out_ref[...] = h + jnp.maximum(h_out, 0.0)


def _conv_mlp(h, agg, W1, b1, W2, b2):
    return pl.pallas_call(
        _conv_mlp_body,
        grid=(N // BN,),
        in_specs=[
            pl.BlockSpec((BN, H), lambda i: (i, 0)),
            pl.BlockSpec((2, BN, H), lambda i: (0, i, 0)),
            pl.BlockSpec((H, H), lambda i: (0, 0)),
            pl.BlockSpec((H,), lambda i: (0,)),
            pl.BlockSpec((H, H), lambda i: (0, 0)),
            pl.BlockSpec((H,), lambda i: (0,)),
        ],
        out_specs=pl.BlockSpec((BN, H), lambda i: (i, 0)),
        out_shape=jax.ShapeDtypeStruct((N, H), jnp.float32),
    )(h, agg, W1, b1, W2, b2)


def kernel(node_emb, node_type, node_degree, pos, linker_mask, fragment_mask,
           edge_index, edge_type, batch, time_step,
           deg_emb, et_emb, el_W1, el_b1, el_W2, el_b2,
           gin_W1, gin_b1, gin_W2, gin_b2,
           gm_W1, gm_b1, gm_W2, gm_b2, gm_W3, gm_b3):
    row = edge_index[0].astype(jnp.int32)
    col = edge_index[1].astype(jnp.int32)
    packed = jnp.bitwise_or(row, jnp.left_shift(col, 16))
    packed3 = jnp.concatenate([packed, jnp.zeros((PAD,), jnp.int32)]
                              ).reshape(2, 16, CH, K)

    d0 = pos[row] - pos[col]
    edge_length = jnp.sqrt(jnp.sum(d0 * d0, axis=-1, keepdims=True) + 1e-12)
    el_pad = jnp.concatenate([edge_length, jnp.zeros((PAD, 1), jnp.float32)])
    et_pad = jnp.concatenate([edge_type.astype(jnp.int32),
                              jnp.zeros((PAD,), jnp.int32)])
    edge_attr = _edge_encoder(el_pad, et_pad, el_W1, el_b1, el_W2, el_b2, et_emb)

    z = jnp.take(deg_emb, node_degree, axis=0)
    W1a = gm_W1[:H]
    W1b = gm_W1[H:2 * H]
    W1c = gm_W1[2 * H:]
    C = edge_attr[:E] @ W1c + gm_b1
    pos0 = pos
    pf = pos
    for b in range(NB):
        h = z
        for c in range(NC):
            agg = _sc_conv_agg(h, edge_attr, packed3)
            h = _conv_mlp(h, agg, gin_W1[b, c], gin_b1[b, c],
                          gin_W2[b, c], gin_b2[b, c])
        A = h @ W1a
        B = h @ W1b
        e1 = jax.nn.relu(A[row] + B[col] + C)
        e2 = jax.nn.relu(e1 @ gm_W2 + gm_b2)
        edge_inv = e2 @ gm_W3 + gm_b3
        dd_dr = (pf[row] - pf[col]) / edge_length
        upd = dd_dr * edge_inv
        node_eq = (jax.ops.segment_sum(upd, row, num_segments=N)
                   - jax.ops.segment_sum(upd, col, num_segments=N))
        pf = pf + node_eq
        pf = pf * linker_mask + pos0 * fragment_mask
    return pf - pos0


# speedup vs baseline: 2.9305x; 1.8141x over previous
import functools

import jax
import jax.numpy as jnp
from jax import lax
from jax.experimental import pallas as pl
from jax.experimental.pallas import tpu as pltpu
import jax.experimental.pallas.tpu_sc as plsc

N = 10000
E = 160000
H = 128
NB = 2
NC = 4

K = 64
CH = 80
E_PAD = 2 * 16 * CH * K
PAD = E_PAD - E
AGG_ROWS = 10240
NEG = -1e9

EBLK = 8192
BN = 2000


def _bf16_dot(a, b):
    return jnp.dot(a.astype(jnp.bfloat16), b.astype(jnp.bfloat16),
                   preferred_element_type=jnp.float32)


_SC_MESH = dict(core_axis_name="c", subcore_axis_name="s")
_SC_PARAMS = pltpu.CompilerParams(use_tc_tiling_on_sc=False)


def _edge_encoder_body(d_ref, et_ref, elW1_ref, elb1_ref, elW2_ref, elb2_ref,
                       etemb_ref, W1c_ref, gmb1_ref, attr_ref, C_ref, el_ref):
    d = d_ref[...]
    el = jnp.sqrt(jnp.sum(d * d, axis=1, keepdims=True) + 1e-12)
    eh = jnp.maximum(el * elW1_ref[...] + elb1_ref[...], 0.0)
    eh = _bf16_dot(eh, elW2_ref[...]) + elb2_ref[...]
    et = et_ref[0, 0]
    onehot = (et[:, None] == lax.broadcasted_iota(jnp.int32, (1, 8), 1)
              ).astype(jnp.float32)
    emb = lax.dot(onehot, etemb_ref[...],
                  precision=lax.Precision.HIGHEST)
    attr = eh * emb
    gid = pl.program_id(0) * EBLK + lax.broadcasted_iota(jnp.int32, (EBLK, 1), 0)
    attr = jnp.where(gid < E, attr, NEG)
    attr_ref[...] = attr
    C_ref[...] = _bf16_dot(attr, W1c_ref[...]) + gmb1_ref[...]
    el_ref[...] = el


def _edge_encoder(d016, et_pad, el_W1, el_b1, el_W2, el_b2, et_emb, W1c, gm_b1):
    nblk = E_PAD // EBLK
    et3 = et_pad.reshape(nblk, 1, EBLK)
    return pl.pallas_call(
        _edge_encoder_body,
        grid=(nblk,),
        in_specs=[
            pl.BlockSpec((EBLK, 16), lambda i: (i, 0)),
            pl.BlockSpec((1, 1, EBLK), lambda i: (i, 0, 0)),
            pl.BlockSpec((1, H), lambda i: (0, 0)),
            pl.BlockSpec((H,), lambda i: (0,)),
            pl.BlockSpec((H, H), lambda i: (0, 0)),
            pl.BlockSpec((H,), lambda i: (0,)),
            pl.BlockSpec((8, H), lambda i: (0, 0)),
            pl.BlockSpec((H, H), lambda i: (0, 0)),
            pl.BlockSpec((H,), lambda i: (0,)),
        ],
        out_specs=[
            pl.BlockSpec((EBLK, H), lambda i: (i, 0)),
            pl.BlockSpec((EBLK, H), lambda i: (i, 0)),
            pl.BlockSpec((EBLK, 1), lambda i: (i, 0)),
        ],
        out_shape=[
            jax.ShapeDtypeStruct((E_PAD, H), jnp.float32),
            jax.ShapeDtypeStruct((E_PAD, H), jnp.float32),
            jax.ShapeDtypeStruct((E_PAD, 1), jnp.float32),
        ],
    )(d016, et3, el_W1, el_b1, el_W2, el_b2, et_emb, W1c, gm_b1)


def _deg_embed_body(nd_ref, emb_ref, z_ref):
    nd = nd_ref[0, 0]
    onehot = (nd[:, None] == lax.broadcasted_iota(jnp.int32, (1, 64), 1)
              ).astype(jnp.float32)
    z_ref[...] = lax.dot(onehot, emb_ref[...], precision=lax.Precision.HIGHEST)


def _deg_embed(node_degree, deg_emb):
    nd3 = node_degree.astype(jnp.int32).reshape(N // BN, 1, BN)
    return pl.pallas_call(
        _deg_embed_body,
        grid=(N // BN,),
        in_specs=[
            pl.BlockSpec((1, 1, BN), lambda i: (i, 0, 0)),
            pl.BlockSpec((64, H), lambda i: (0, 0)),
        ],
        out_specs=pl.BlockSpec((BN, H), lambda i: (i, 0)),
        out_shape=jax.ShapeDtypeStruct((N, H), jnp.float32),
    )(nd3, deg_emb)


def _sc_conv_agg(h, attr_pad, packed3):
    @functools.partial(
        pl.kernel,
        out_type=jax.ShapeDtypeStruct((2, AGG_ROWS, H), jnp.float32),
        mesh=plsc.VectorSubcoreMesh(**_SC_MESH),
        scratch_types=[
            pltpu.VMEM((CH, K), jnp.int32),
            pltpu.VMEM((2, K), jnp.int32),
            pltpu.VMEM((2, K), jnp.int32),
            pltpu.VMEM((2, K, H), jnp.float32),
            pltpu.VMEM((2, K, H), jnp.float32),
            pltpu.VMEM_SHARED((AGG_ROWS, H), jnp.float32),
            pltpu.SemaphoreType.DMA,
            pltpu.SemaphoreType.DMA,
            pltpu.SemaphoreType.DMA,
            pltpu.SemaphoreType.DMA,
        ],
    )
    def k(h_hbm, attr_hbm, packed_hbm, out_hbm,
          packed_v, idx_r, idx_c, rows_v, attr_v, agg_s, g0, g1, a0, a1):
        cid = lax.axis_index("c")
        sid = lax.axis_index("s")
        wbase = (cid * 16 + sid) * CH
        sem_g = (g0, g1)
        sem_a = (a0, a1)
        zero = jnp.zeros((16,), jnp.float32)

        def zbody(i, carry):
            for l in range(8):
                rows_v[0, i, pl.ds(l * 16, 16)] = zero
            return carry
        lax.fori_loop(0, K, zbody, 0)
        for kk in range(640 // K):
            pltpu.sync_copy(rows_v.at[0], agg_s.at[pl.ds(sid * 640 + kk * K, K)])
        plsc.subcore_barrier()

        pltpu.sync_copy(packed_hbm.at[cid, sid], packed_v)

        def start(j, b):
            for l in range(K // 16):
                s = pl.ds(l * 16, 16)
                p = packed_v[j, s]
                idx_r[b, s] = jnp.bitwise_and(p, 0xFFFF)
                idx_c[b, s] = lax.shift_right_logical(p, 16)
            pltpu.async_copy(h_hbm.at[idx_r.at[b]], rows_v.at[b], sem_g[b])
            pltpu.async_copy(attr_hbm.at[pl.ds((wbase + j) * K, K)],
                             attr_v.at[b], sem_a[b])

        start(0, 0)
        start(1, 1)

        def body(j2, carry):
            for b in range(2):
                j = j2 * 2 + b
                pltpu.make_async_copy(h_hbm.at[idx_r.at[b]],
                                      rows_v.at[b], sem_g[b]).wait()
                pltpu.make_async_copy(attr_hbm.at[pl.ds((wbase + j) * K, K)],
                                      attr_v.at[b], sem_a[b]).wait()

                def mbody(i, c2):
                    for l in range(8):
                        s = pl.ds(l * 16, 16)
                        rows_v[b, i, s] = jnp.maximum(
                            rows_v[b, i, s] + attr_v[b, i, s], 0.0)
                    return c2
                lax.fori_loop(0, K, mbody, 0)
                pltpu.sync_copy(rows_v.at[b], agg_s.at[idx_c.at[b]], add=True)

                @pl.when(j + 2 < CH)
                def _():
                    start(j + 2, b)
            return carry
        lax.fori_loop(0, CH // 2, body, 0)
        plsc.subcore_barrier()
        pltpu.sync_copy(agg_s.at[pl.ds(sid * 640, 640)],
                        out_hbm.at[cid, pl.ds(sid * 640, 640)])

    return k(h, attr_pad, packed3)


def _sc_pos_diff(pf16, packed3):
    @functools.partial(
        pl.kernel,
        out_type=jax.ShapeDtypeStruct((E_PAD, 16), jnp.float32),
        mesh=plsc.VectorSubcoreMesh(**_SC_MESH),
        compiler_params=_SC_PARAMS,
        scratch_types=[
            pltpu.VMEM((CH, K), jnp.int32),
            pltpu.VMEM((2, K), jnp.int32),
            pltpu.VMEM((2, K), jnp.int32),
            pltpu.VMEM((2, K, 16), jnp.float32),
            pltpu.VMEM((2, K, 16), jnp.float32),
            pltpu.SemaphoreType.DMA,
            pltpu.SemaphoreType.DMA,
            pltpu.SemaphoreType.DMA,
            pltpu.SemaphoreType.DMA,
        ],
    )
    def k(pf_hbm, packed_hbm, out_hbm,
          packed_v, idx_r, idx_c, a_v, b_v, g0, g1, h0, h1):
        cid = lax.axis_index("c")
        sid = lax.axis_index("s")
        wbase = (cid * 16 + sid) * CH
        sem_a = (g0, g1)
        sem_b = (h0, h1)
        pltpu.sync_copy(packed_hbm.at[cid, sid], packed_v)

        def start(j, b):
            for l in range(K // 16):
                s = pl.ds(l * 16, 16)
                p = packed_v[j, s]
                idx_r[b, s] = jnp.bitwise_and(p, 0xFFFF)
                idx_c[b, s] = lax.shift_right_logical(p, 16)
            pltpu.async_copy(pf_hbm.at[idx_r.at[b]], a_v.at[b], sem_a[b])
            pltpu.async_copy(pf_hbm.at[idx_c.at[b]], b_v.at[b], sem_b[b])

        start(0, 0)
        start(1, 1)

        def body(j2, carry):
            for b in range(2):
                j = j2 * 2 + b
                pltpu.make_async_copy(pf_hbm.at[idx_r.at[b]],
                                      a_v.at[b], sem_a[b]).wait()
                pltpu.make_async_copy(pf_hbm.at[idx_c.at[b]],
                                      b_v.at[b], sem_b[b]).wait()

                def mbody(i, c2):
                    s = pl.ds(0, 16)
                    a_v[b, i, s] = a_v[b, i, s] - b_v[b, i, s]
                    return c2
                lax.fori_loop(0, K, mbody, 0)
                pltpu.sync_copy(a_v.at[b], out_hbm.at[pl.ds((wbase + j) * K, K)])

                @pl.when(j + 2 < CH)
                def _():
                    start(j + 2, b)
            return carry
        lax.fori_loop(0, CH // 2, body, 0)

    return k(pf16, packed3)


def _sc_pair(A, Bm, C, packed3):
    @functools.partial(
        pl.kernel,
        out_type=jax.ShapeDtypeStruct((E_PAD, H), jnp.float32),
        mesh=plsc.VectorSubcoreMesh(**_SC_MESH),
        scratch_types=[
            pltpu.VMEM((CH, K), jnp.int32),
            pltpu.VMEM((2, K), jnp.int32),
            pltpu.VMEM((2, K), jnp.int32),
            pltpu.VMEM((2, K, H), jnp.float32),
            pltpu.VMEM((2, K, H), jnp.float32),
            pltpu.VMEM((2, K, H), jnp.float32),
            pltpu.SemaphoreType.DMA,
            pltpu.SemaphoreType.DMA,
            pltpu.SemaphoreType.DMA,
            pltpu.SemaphoreType.DMA,
            pltpu.SemaphoreType.DMA,
            pltpu.SemaphoreType.DMA,
        ],
    )
    def k(A_hbm, B_hbm, C_hbm, packed_hbm, out_hbm,
          packed_v, idx_r, idx_c, a_v, b_v, c_v, s0, s1, s2, s3, s4, s5):
        cid = lax.axis_index("c")
        sid = lax.axis_index("s")
        wbase = (cid * 16 + sid) * CH
        sem_a = (s0, s1)
        sem_b = (s2, s3)
        sem_c = (s4, s5)
        pltpu.sync_copy(packed_hbm.at[cid, sid], packed_v)

        def start(j, b):
            for l in range(K // 16):
                s = pl.ds(l * 16, 16)
                p = packed_v[j, s]
                idx_r[b, s] = jnp.bitwise_and(p, 0xFFFF)
                idx_c[b, s] = lax.shift_right_logical(p, 16)
            pltpu.async_copy(A_hbm.at[idx_r.at[b]], a_v.at[b], sem_a[b])
            pltpu.async_copy(B_hbm.at[idx_c.at[b]], b_v.at[b], sem_b[b])
            pltpu.async_copy(C_hbm.at[pl.ds((wbase + j) * K, K)],
                             c_v.at[b], sem_c[b])

        start(0, 0)
        start(1, 1)

        def body(j2, carry):
            for b in range(2):
                j = j2 * 2 + b
                pltpu.make_async_copy(A_hbm.at[idx_r.at[b]],
                                      a_v.at[b], sem_a[b]).wait()
                pltpu.make_async_copy(B_hbm.at[idx_c.at[b]],
                                      b_v.at[b], sem_b[b]).wait()
                pltpu.make_async_copy(C_hbm.at[pl.ds((wbase + j) * K, K)],
                                      c_v.at[b], sem_c[b]).wait()

                def mbody(i, c2):
                    for l in range(8):
                        s = pl.ds(l * 16, 16)
                        a_v[b, i, s] = jnp.maximum(
                            a_v[b, i, s] + b_v[b, i, s] + c_v[b, i, s], 0.0)
                    return c2
                lax.fori_loop(0, K, mbody, 0)
                pltpu.sync_copy(a_v.at[b], out_hbm.at[pl.ds((wbase + j) * K, K)])

                @pl.when(j + 2 < CH)
                def _():
                    start(j + 2, b)
            return carry
        lax.fori_loop(0, CH // 2, body, 0)

    return k(A, Bm, C, packed3)


def _sc_eq(pf16, iol16, packed3):
    @functools.partial(
        pl.kernel,
        out_type=jax.ShapeDtypeStruct((2, AGG_ROWS, 16), jnp.float32),
        mesh=plsc.VectorSubcoreMesh(**_SC_MESH),
        compiler_params=_SC_PARAMS,
        scratch_types=[
            pltpu.VMEM((CH, K), jnp.int32),
            pltpu.VMEM((2, K), jnp.int32),
            pltpu.VMEM((2, K), jnp.int32),
            pltpu.VMEM((2, K, 16), jnp.float32),
            pltpu.VMEM((2, K, 16), jnp.float32),
            pltpu.VMEM((2, K, 16), jnp.float32),
            pltpu.VMEM_SHARED((AGG_ROWS, 16), jnp.float32),
            pltpu.SemaphoreType.DMA,
            pltpu.SemaphoreType.DMA,
            pltpu.SemaphoreType.DMA,
            pltpu.SemaphoreType.DMA,
            pltpu.SemaphoreType.DMA,
            pltpu.SemaphoreType.DMA,
        ],
    )
    def k(pf_hbm, iol_hbm, packed_hbm, out_hbm,
          packed_v, idx_r, idx_c, a_v, b_v, i_v, acc_s, s0, s1, s2, s3, s4, s5):
        cid = lax.axis_index("c")
        sid = lax.axis_index("s")
        wbase = (cid * 16 + sid) * CH
        sem_a = (s0, s1)
        sem_b = (s2, s3)
        sem_i = (s4, s5)
        zero = jnp.zeros((16,), jnp.float32)

        def zbody(i, carry):
            a_v[0, i, pl.ds(0, 16)] = zero
            return carry
        lax.fori_loop(0, K, zbody, 0)
        for kk in range(640 // K):
            pltpu.sync_copy(a_v.at[0], acc_s.at[pl.ds(sid * 640 + kk * K, K)])
        plsc.subcore_barrier()
        pltpu.sync_copy(packed_hbm.at[cid, sid], packed_v)

        def start(j, b):
            for l in range(K // 16):
                s = pl.ds(l * 16, 16)
                p = packed_v[j, s]
                idx_r[b, s] = jnp.bitwise_and(p, 0xFFFF)
                idx_c[b, s] = lax.shift_right_logical(p, 16)
            pltpu.async_copy(pf_hbm.at[idx_r.at[b]], a_v.at[b], sem_a[b])
            pltpu.async_copy(pf_hbm.at[idx_c.at[b]], b_v.at[b], sem_b[b])
            pltpu.async_copy(iol_hbm.at[pl.ds((wbase + j) * K, K)],
                             i_v.at[b], sem_i[b])

        start(0, 0)
        start(1, 1)

        def body(j2, carry):
            for b in range(2):
                j = j2 * 2 + b
                pltpu.make_async_copy(pf_hbm.at[idx_r.at[b]],
                                      a_v.at[b], sem_a[b]).wait()
                pltpu.make_async_copy(pf_hbm.at[idx_c.at[b]],
                                      b_v.at[b], sem_b[b]).wait()
                pltpu.make_async_copy(iol_hbm.at[pl.ds((wbase + j) * K, K)],
                                      i_v.at[b], sem_i[b]).wait()

                def mbody(i, c2):
                    s = pl.ds(0, 16)
                    u = (a_v[b, i, s] - b_v[b, i, s]) * i_v[b, i, s]
                    a_v[b, i, s] = u
                    b_v[b, i, s] = -u
                    return c2
                lax.fori_loop(0, K, mbody, 0)
                pltpu.sync_copy(a_v.at[b], acc_s.at[idx_r.at[b]], add=True)
                pltpu.sync_copy(b_v.at[b], acc_s.at[idx_c.at[b]], add=True)

                @pl.when(j + 2 < CH)
                def _():
                    start(j + 2, b)
            return carry
        lax.fori_loop(0, CH // 2, body, 0)
        plsc.subcore_barrier()
        pltpu.sync_copy(acc_s.at[pl.ds(sid * 640, 640)],
                        out_hbm.at[cid, pl.ds(sid * 640, 640)])

    return k(pf16, iol16, packed3)


def _conv_mlp_body(h_ref, agg_ref, W1_ref, b1_ref, W2_ref, b2_ref, out_ref):
    h = h_ref[...]
    h_in = h + (agg_ref[0] + agg_ref[1])
    t = jnp.maximum(_bf16_dot(h_in, W1_ref[...]) + b1_ref[...], 0.0)
    h_out = _bf16_dot(t, W2_ref[...]) + b2_ref[...]
    out_ref[...] = h + jnp.maximum(h_out, 0.0)


def _conv_mlp(h, agg, W1, b1, W2, b2):
    return pl.pallas_call(
        _conv_mlp_body,
        grid=(N // BN,),
        in_specs=[
            pl.BlockSpec((BN, H), lambda i: (i, 0)),
            pl.BlockSpec((2, BN, H), lambda i: (0, i, 0)),
            pl.BlockSpec((H, H), lambda i: (0, 0)),
            pl.BlockSpec((H,), lambda i: (0,)),
            pl.BlockSpec((H, H), lambda i: (0, 0)),
            pl.BlockSpec((H,), lambda i: (0,)),
        ],
        out_specs=pl.BlockSpec((BN, H), lambda i: (i, 0)),
        out_shape=jax.ShapeDtypeStruct((N, H), jnp.float32),
    )(h, agg, W1, b1, W2, b2)


def _conv_mlp_ab_body(h_ref, agg_ref, W1_ref, b1_ref, W2_ref, b2_ref,
                      W1a_ref, W1b_ref, out_ref, A_ref, B_ref):
    h = h_ref[...]
    h_in = h + (agg_ref[0] + agg_ref[1])
    t = jnp.maximum(_bf16_dot(h_in, W1_ref[...]) + b1_ref[...], 0.0)
    h_out = _bf16_dot(t, W2_ref[...]) + b2_ref[...]
    hn = h + jnp.maximum(h_out, 0.0)
    out_ref[...] = hn
    A_ref[...] = _bf16_dot(hn, W1a_ref[...])
    B_ref[...] = _bf16_dot(hn, W1b_ref[...])


def _conv_mlp_ab(h, agg, W1, b1, W2, b2, W1a, W1b):
    return pl.pallas_call(
        _conv_mlp_ab_body,
        grid=(N // BN,),
        in_specs=[
            pl.BlockSpec((BN, H), lambda i: (i, 0)),
            pl.BlockSpec((2, BN, H), lambda i: (0, i, 0)),
            pl.BlockSpec((H, H), lambda i: (0, 0)),
            pl.BlockSpec((H,), lambda i: (0,)),
            pl.BlockSpec((H, H), lambda i: (0, 0)),
            pl.BlockSpec((H,), lambda i: (0,)),
            pl.BlockSpec((H, H), lambda i: (0, 0)),
            pl.BlockSpec((H, H), lambda i: (0, 0)),
        ],
        out_specs=[
            pl.BlockSpec((BN, H), lambda i: (i, 0)),
            pl.BlockSpec((BN, H), lambda i: (i, 0)),
            pl.BlockSpec((BN, H), lambda i: (i, 0)),
        ],
        out_shape=[
            jax.ShapeDtypeStruct((N, H), jnp.float32),
            jax.ShapeDtypeStruct((N, H), jnp.float32),
            jax.ShapeDtypeStruct((N, H), jnp.float32),
        ],
    )(h, agg, W1, b1, W2, b2, W1a, W1b)


def _inv_body(e1_ref, el_ref, W2_ref, b2_ref, W3_ref, b3_ref, iol_ref):
    e2 = jnp.maximum(_bf16_dot(e1_ref[...], W2_ref[...]) + b2_ref[...], 0.0)
    inv = _bf16_dot(e2, W3_ref[...]) + b3_ref[...]
    iol = inv / el_ref[...]
    iol_ref[...] = jnp.broadcast_to(iol, (EBLK, 16))


def _inv_mlp(e1, el, gm_W2, gm_b2, gm_W3, gm_b3):
    return pl.pallas_call(
        _inv_body,
        grid=(E_PAD // EBLK,),
        in_specs=[
            pl.BlockSpec((EBLK, H), lambda i: (i, 0)),
            pl.BlockSpec((EBLK, 1), lambda i: (i, 0)),
            pl.BlockSpec((H, H // 2), lambda i: (0, 0)),
            pl.BlockSpec((H // 2,), lambda i: (0,)),
            pl.BlockSpec((H // 2, 1), lambda i: (0, 0)),
            pl.BlockSpec((1,), lambda i: (0,)),
        ],
        out_specs=pl.BlockSpec((EBLK, 16), lambda i: (i, 0)),
        out_shape=jax.ShapeDtypeStruct((E_PAD, 16), jnp.float32),
    )(e1, el, gm_W2, gm_b2, gm_W3, gm_b3)


def _pos_body(pf_ref, eq_ref, lm_ref, fm_ref, p0_ref, out_ref):
    pf = pf_ref[...] + (eq_ref[0] + eq_ref[1])
    out_ref[...] = pf * lm_ref[...] + p0_ref[...] * fm_ref[...]


def _pos_update(pf16, eqp, linker_mask, fragment_mask, pos016):
    return pl.pallas_call(
        _pos_body,
        grid=(N // BN,),
        in_specs=[
            pl.BlockSpec((BN, 16), lambda i: (i, 0)),
            pl.BlockSpec((2, BN, 16), lambda i: (0, i, 0)),
            pl.BlockSpec((BN, 1), lambda i: (i, 0)),
            pl.BlockSpec((BN, 1), lambda i: (i, 0)),
            pl.BlockSpec((BN, 16), lambda i: (i, 0)),
        ],
        out_specs=pl.BlockSpec((BN, 16), lambda i: (i, 0)),
        out_shape=jax.ShapeDtypeStruct((N, 16), jnp.float32),
    )(pf16, eqp, linker_mask, fragment_mask, pos016)


def kernel(node_emb, node_type, node_degree, pos, linker_mask, fragment_mask,
           edge_index, edge_type, batch, time_step,
           deg_emb, et_emb, el_W1, el_b1, el_W2, el_b2,
           gin_W1, gin_b1, gin_W2, gin_b2,
           gm_W1, gm_b1, gm_W2, gm_b2, gm_W3, gm_b3):
    row = edge_index[0].astype(jnp.int32)
    col = edge_index[1].astype(jnp.int32)
    packed = jnp.bitwise_or(row, jnp.left_shift(col, 16))
    packed3 = jnp.concatenate([packed, jnp.zeros((PAD,), jnp.int32)]
                              ).reshape(2, 16, CH, K)
    et_pad = jnp.concatenate([edge_type.astype(jnp.int32),
                              jnp.zeros((PAD,), jnp.int32)])
    pos016 = jnp.pad(pos, ((0, 0), (0, 13)))

    d016 = _sc_pos_diff(pos016, packed3)
    edge_attr, C, el = _edge_encoder(d016, et_pad, el_W1, el_b1, el_W2, el_b2,
                                     et_emb, gm_W1[2 * H:], gm_b1)
    z = _deg_embed(node_degree, deg_emb)

    pf16 = pos016
    for b in range(NB):
        h = z
        for c in range(NC):
            agg = _sc_conv_agg(h, edge_attr, packed3)
            if c < NC - 1:
                h = _conv_mlp(h, agg, gin_W1[b, c], gin_b1[b, c],
                              gin_W2[b, c], gin_b2[b, c])
            else:
                h, A, Bm = _conv_mlp_ab(h, agg, gin_W1[b, c], gin_b1[b, c],
                                        gin_W2[b, c], gin_b2[b, c],
                                        gm_W1[:H], gm_W1[H:2 * H])
        e1 = _sc_pair(A, Bm, C, packed3)
        iol16 = _inv_mlp(e1, el, gm_W2, gm_b2, gm_W3, gm_b3)
        eqp = _sc_eq(pf16, iol16, packed3)
        pf16 = _pos_update(pf16, eqp, linker_mask, fragment_mask, pos016)
    return (pf16 - pos016)[:, :3]
